# Initial kernel scaffold; baseline (speedup 1.0000x reference)
import jax
import jax.numpy as jnp
from jax.experimental import pallas as pl


def kernel(hidden_states, gate_w, gate_proj_w, up_proj_w, down_proj_w):
    raise NotImplementedError("write your pallas kernel here")


# speedup vs baseline: 1.4785x; 1.4785x over previous
import functools

import jax
import jax.numpy as jnp
from jax.experimental import pallas as pl
from jax.experimental.pallas import tpu as pltpu

HIDDEN = 2048
FF = 2048
E = 8
BT = 1024
FB = 512


def _moe_body(x_ref, gw_ref, wg_ref, wu_ref, wd_ref,
              out_ref, logits_ref, probs_ref):
    e = pl.program_id(1)
    f = pl.program_id(2)

    @pl.when((e == 0) & (f == 0))
    def _router():
        logits = jnp.dot(gw_ref[...].astype(jnp.bfloat16), x_ref[...],
                         preferred_element_type=jnp.float32)
        logits_ref[...] = logits
        m = jnp.max(logits, axis=0, keepdims=True)
        p = jnp.exp(logits - m)
        probs_ref[...] = p / jnp.sum(p, axis=0, keepdims=True)
        out_ref[...] = jnp.zeros_like(out_ref)

    xb = x_ref[...]
    wg = wg_ref[0].astype(jnp.bfloat16)
    wu = wu_ref[0].astype(jnp.bfloat16)
    gate = jnp.dot(wg, xb, preferred_element_type=jnp.float32)
    up = jnp.dot(wu, xb, preferred_element_type=jnp.float32)
    w_e = probs_ref[pl.ds(e, 1), :]
    inter = (jax.nn.silu(gate) * up * w_e).astype(jnp.bfloat16)
    wd = wd_ref[0].astype(jnp.bfloat16)
    out_ref[...] += jnp.dot(wd, inter, preferred_element_type=jnp.float32)


@functools.partial(jax.jit, static_argnums=())
def kernel(hidden_states, gate_w, gate_proj_w, up_proj_w, down_proj_w):
    b, s, h = hidden_states.shape
    t = b * s
    x = hidden_states.reshape(t, h).T.astype(jnp.bfloat16)

    nt = t // BT
    nf = FF // FB
    grid = (nt, E, nf)

    out_t, logits_t = pl.pallas_call(
        _moe_body,
        grid=grid,
        in_specs=[
            pl.BlockSpec((h, BT), lambda ti, ei, fi: (0, ti)),
            pl.BlockSpec((E, h), lambda ti, ei, fi: (0, 0)),
            pl.BlockSpec((1, FB, h), lambda ti, ei, fi: (ei, fi, 0)),
            pl.BlockSpec((1, FB, h), lambda ti, ei, fi: (ei, fi, 0)),
            pl.BlockSpec((1, h, FB), lambda ti, ei, fi: (ei, 0, fi)),
        ],
        out_specs=[
            pl.BlockSpec((h, BT), lambda ti, ei, fi: (0, ti)),
            pl.BlockSpec((E, BT), lambda ti, ei, fi: (0, ti)),
        ],
        out_shape=[
            jax.ShapeDtypeStruct((h, t), jnp.float32),
            jax.ShapeDtypeStruct((E, t), jnp.float32),
        ],
        scratch_shapes=[pltpu.VMEM((E, BT), jnp.float32)],
        compiler_params=pltpu.CompilerParams(
            dimension_semantics=("parallel", "arbitrary", "arbitrary"),
        ),
    )(x, gate_w, gate_proj_w, up_proj_w, down_proj_w)

    final = out_t.T.reshape(b, s, h)
    return final, logits_t.T


# speedup vs baseline: 1.4816x; 1.0021x over previous
import functools

import jax
import jax.numpy as jnp
from jax import lax
from jax.experimental import pallas as pl
from jax.experimental.pallas import tpu as pltpu

HIDDEN = 2048
FF = 2048
E = 8
BT = 1024
FB = 512

_DIMS = (((1,), (0,)), ((), ()))


def _mm(a, b):
    return lax.dot_general(a, b, _DIMS, precision=lax.Precision.DEFAULT,
                           preferred_element_type=jnp.float32)


def _moe_body(x_ref, gw_ref, wg_ref, wu_ref, wd_ref,
              out_ref, logits_ref, probs_ref):
    e = pl.program_id(1)
    f = pl.program_id(2)

    @pl.when((e == 0) & (f == 0))
    def _router():
        logits = _mm(gw_ref[...], x_ref[...])
        logits_ref[...] = logits
        m = jnp.max(logits, axis=0, keepdims=True)
        p = jnp.exp(logits - m)
        probs_ref[...] = p / jnp.sum(p, axis=0, keepdims=True)
        out_ref[...] = jnp.zeros_like(out_ref)

    xb = x_ref[...]
    gate = _mm(wg_ref[0], xb)
    up = _mm(wu_ref[0], xb)
    w_e = probs_ref[pl.ds(e, 1), :]
    inter = (jax.nn.silu(gate) * up * w_e).astype(jnp.bfloat16)
    out_ref[...] += _mm(wd_ref[0], inter)


@functools.partial(jax.jit, static_argnums=())
def kernel(hidden_states, gate_w, gate_proj_w, up_proj_w, down_proj_w):
    b, s, h = hidden_states.shape
    t = b * s
    x = hidden_states.reshape(t, h).T.astype(jnp.bfloat16)

    nt = t // BT
    nf = FF // FB
    grid = (nt, E, nf)

    out_t, logits_t = pl.pallas_call(
        _moe_body,
        grid=grid,
        in_specs=[
            pl.BlockSpec((h, BT), lambda ti, ei, fi: (0, ti)),
            pl.BlockSpec((E, h), lambda ti, ei, fi: (0, 0)),
            pl.BlockSpec((1, FB, h), lambda ti, ei, fi: (ei, fi, 0)),
            pl.BlockSpec((1, FB, h), lambda ti, ei, fi: (ei, fi, 0)),
            pl.BlockSpec((1, h, FB), lambda ti, ei, fi: (ei, 0, fi)),
        ],
        out_specs=[
            pl.BlockSpec((h, BT), lambda ti, ei, fi: (0, ti)),
            pl.BlockSpec((E, BT), lambda ti, ei, fi: (0, ti)),
        ],
        out_shape=[
            jax.ShapeDtypeStruct((h, t), jnp.float32),
            jax.ShapeDtypeStruct((E, t), jnp.float32),
        ],
        scratch_shapes=[pltpu.VMEM((E, BT), jnp.float32)],
        compiler_params=pltpu.CompilerParams(
            dimension_semantics=("parallel", "arbitrary", "arbitrary"),
        ),
    )(x, gate_w, gate_proj_w, up_proj_w, down_proj_w)

    final = out_t.T.reshape(b, s, h)
    return final, logits_t.T


# speedup vs baseline: 1.5780x; 1.0651x over previous
import functools

import jax
import jax.numpy as jnp
from jax import lax
from jax.experimental import pallas as pl
from jax.experimental.pallas import tpu as pltpu

HIDDEN = 2048
FF = 2048
E = 8
BT = 1024
FB = 512


def _mm(a, b, dims):
    return lax.dot_general(a, b, (dims, ((), ())),
                           precision=lax.Precision.DEFAULT,
                           preferred_element_type=jnp.float32)


def _moe_body(x_ref, gw_ref, wg_ref, wu_ref, wd_ref,
              out_ref, logits_ref, probs_ref):
    e = pl.program_id(1)
    f = pl.program_id(2)

    @pl.when((e == 0) & (f == 0))
    def _router():
        logits = _mm(gw_ref[...], x_ref[...], ((1,), (0,)))
        logits_ref[...] = logits
        m = jnp.max(logits, axis=0, keepdims=True)
        p = jnp.exp(logits - m)
        probs_ref[...] = p / jnp.sum(p, axis=0, keepdims=True)
        out_ref[...] = jnp.zeros_like(out_ref)

    xb = x_ref[...]
    gate = _mm(wg_ref[0], xb, ((1,), (0,)))
    up = _mm(wu_ref[0], xb, ((1,), (0,)))
    w_e = probs_ref[pl.ds(e, 1), :]
    inter = (jax.nn.silu(gate) * up * w_e).astype(jnp.bfloat16)
    out_ref[...] += _mm(inter, wd_ref[0], ((0,), (1,)))


@functools.partial(jax.jit, static_argnums=())
def kernel(hidden_states, gate_w, gate_proj_w, up_proj_w, down_proj_w):
    b, s, h = hidden_states.shape
    t = b * s
    x = hidden_states.reshape(t, h).T.astype(jnp.bfloat16)

    nt = t // BT
    nf = FF // FB
    grid = (nt, E, nf)

    out, logits_t = pl.pallas_call(
        _moe_body,
        grid=grid,
        in_specs=[
            pl.BlockSpec((h, BT), lambda ti, ei, fi: (0, ti)),
            pl.BlockSpec((E, h), lambda ti, ei, fi: (0, 0)),
            pl.BlockSpec((1, FB, h), lambda ti, ei, fi: (ei, fi, 0)),
            pl.BlockSpec((1, FB, h), lambda ti, ei, fi: (ei, fi, 0)),
            pl.BlockSpec((1, h, FB), lambda ti, ei, fi: (ei, 0, fi)),
        ],
        out_specs=[
            pl.BlockSpec((BT, h), lambda ti, ei, fi: (ti, 0)),
            pl.BlockSpec((E, BT), lambda ti, ei, fi: (0, ti)),
        ],
        out_shape=[
            jax.ShapeDtypeStruct((t, h), jnp.float32),
            jax.ShapeDtypeStruct((E, t), jnp.float32),
        ],
        scratch_shapes=[pltpu.VMEM((E, BT), jnp.float32)],
        compiler_params=pltpu.CompilerParams(
            dimension_semantics=("parallel", "arbitrary", "arbitrary"),
        ),
    )(x, gate_w, gate_proj_w, up_proj_w, down_proj_w)

    final = out.reshape(b, s, h)
    return final, logits_t.T
